# baseline (speedup 1.0000x reference)
import functools
import math

import jax
import jax.numpy as jnp
from jax import lax
from jax.experimental import pallas as pl
from jax.experimental.pallas import tpu as pltpu
from jax.experimental.pallas import tpu_sc as plsc

_NUM_CORES = 2
_NUM_SUBCORES = 16
_NUM_WORKERS = _NUM_CORES * _NUM_SUBCORES
_LANES = 16


@functools.lru_cache(maxsize=None)
def _make_gather(B, V, D, scale):
  assert B % _NUM_WORKERS == 0
  b_per_w = B // _NUM_WORKERS
  C = 800
  assert b_per_w % C == 0 and C % 8 == 0
  n_chunks = b_per_w // C

  mesh = plsc.VectorSubcoreMesh(core_axis_name="c", subcore_axis_name="s")

  @functools.partial(
      pl.kernel,
      mesh=mesh,
      out_type=jax.ShapeDtypeStruct((B, D), jnp.float32),
      scratch_types=[
          pltpu.VMEM((C,), jnp.int32),
          pltpu.VMEM((C, D), jnp.float32),
          pltpu.SemaphoreType.DMA,
      ],
      compiler_params=pltpu.CompilerParams(use_tc_tiling_on_sc=False),
  )
  def gather_kernel(table_hbm, idx_hbm, out_hbm, idx_v, rows_v, sem):
    wid = lax.axis_index("s") * _NUM_CORES + lax.axis_index("c")
    base = wid * b_per_w

    def do_chunk(g, carry):
      off = base + g * C
      pltpu.sync_copy(idx_hbm.at[pl.ds(off, C)], idx_v)
      pltpu.async_copy(table_hbm.at[idx_v], rows_v, sem).wait()

      def scale_row(i, c2):
        for j in range(D // _LANES):
          sl = pl.ds(j * _LANES, _LANES)
          rows_v[i, sl] = rows_v[i, sl] * scale
        return c2

      lax.fori_loop(0, C, scale_row, 0)
      pltpu.sync_copy(rows_v, out_hbm.at[pl.ds(off, C)])
      return carry

    lax.fori_loop(0, n_chunks, do_chunk, 0)

  return gather_kernel


def kernel(tokens, table):
  bsz, hist = tokens.shape
  vocab, emb = table.shape
  scale = float(math.sqrt(emb))
  flat = tokens.reshape(bsz * hist).astype(jnp.int32)
  out = _make_gather(bsz * hist, vocab, emb, scale)(table, flat)
  return out.reshape(bsz, hist, emb)


# speedup vs baseline: 1.1117x; 1.1117x over previous
import functools
import math

import jax
import jax.numpy as jnp
from jax import lax
from jax.experimental import pallas as pl
from jax.experimental.pallas import tpu as pltpu
from jax.experimental.pallas import tpu_sc as plsc

_NUM_CORES = 2
_NUM_SUBCORES = 16
_NUM_WORKERS = _NUM_CORES * _NUM_SUBCORES
_LANES = 16


@functools.lru_cache(maxsize=None)
def _make_gather(B, V, D, scale):
  assert B % _NUM_WORKERS == 0
  b_per_w = B // _NUM_WORKERS
  C = 800
  n_chunks = b_per_w // C
  assert b_per_w % C == 0 and C % 8 == 0 and n_chunks % 2 == 0

  mesh = plsc.VectorSubcoreMesh(core_axis_name="c", subcore_axis_name="s")

  @functools.partial(
      pl.kernel,
      mesh=mesh,
      out_type=jax.ShapeDtypeStruct((B, D), jnp.float32),
      scratch_types=[
          pltpu.VMEM((C,), jnp.int32),
          pltpu.VMEM((C,), jnp.int32),
          pltpu.VMEM((C, D), jnp.float32),
          pltpu.VMEM((C, D), jnp.float32),
          pltpu.SemaphoreType.DMA,
          pltpu.SemaphoreType.DMA,
          pltpu.SemaphoreType.DMA,
          pltpu.SemaphoreType.DMA,
      ],
      compiler_params=pltpu.CompilerParams(use_tc_tiling_on_sc=False),
  )
  def gather_kernel(table_hbm, idx_hbm, out_hbm, idx0, idx1, rows0, rows1,
                    gsem0, gsem1, ssem0, ssem1):
    wid = lax.axis_index("s") * _NUM_CORES + lax.axis_index("c")
    base = wid * b_per_w
    idx_b = (idx0, idx1)
    rows_b = (rows0, rows1)
    gsem_b = (gsem0, gsem1)
    ssem_b = (ssem0, ssem1)

    def fetch(g, b):
      off = base + g * C
      pltpu.sync_copy(idx_hbm.at[pl.ds(off, C)], idx_b[b])
      pltpu.async_copy(table_hbm.at[idx_b[b]], rows_b[b], gsem_b[b])

    def scale_store(g, b):
      pltpu.make_async_copy(table_hbm.at[idx_b[b]], rows_b[b],
                            gsem_b[b]).wait()

      @plsc.parallel_loop(0, C, step=1, unroll=8)
      def _(i):
        for j in range(D // _LANES):
          sl = pl.ds(j * _LANES, _LANES)
          rows_b[b][i, sl] = rows_b[b][i, sl] * scale

      off = base + g * C
      pltpu.async_copy(rows_b[b], out_hbm.at[pl.ds(off, C)], ssem_b[b])

    def wait_store(g, b):
      off = base + g * C
      pltpu.make_async_copy(rows_b[b], out_hbm.at[pl.ds(off, C)],
                            ssem_b[b]).wait()

    fetch(0, 0)

    def do_pair(p, carry):
      g0 = p * 2

      @pl.when(p > 0)
      def _():
        wait_store(g0 - 1, 1)

      fetch(g0 + 1, 1)
      scale_store(g0, 0)

      @pl.when(g0 + 2 < n_chunks)
      def _():
        wait_store(g0, 0)
        fetch(g0 + 2, 0)

      scale_store(g0 + 1, 1)
      return carry

    lax.fori_loop(0, n_chunks // 2, do_pair, 0)

    wait_store(n_chunks - 2, 0)
    wait_store(n_chunks - 1, 1)

  return gather_kernel


def kernel(tokens, table):
  bsz, hist = tokens.shape
  vocab, emb = table.shape
  scale = float(math.sqrt(emb))
  flat = tokens.reshape(bsz * hist).astype(jnp.int32)
  out = _make_gather(bsz * hist, vocab, emb, scale)(table, flat)
  return out.reshape(bsz, hist, emb)
